# Initial kernel scaffold; baseline (speedup 1.0000x reference)
import jax
import jax.numpy as jnp
from jax.experimental import pallas as pl


def kernel(x, linear_w, linear_b, emb, ffm, W1, b1, g1, be1, m1, v1, W2, b2, g2, be2, m2, v2, Wf1, bf1, gf1, bef1, mf1, vf1, Wf2, bf2, gf2, bef2, mf2, vf2, Wout, bout):
    raise NotImplementedError("write your pallas kernel here")


# speedup vs baseline: 5.9059x; 5.9059x over previous
import functools

import numpy as np
import jax
import jax.numpy as jnp
from jax import lax
from jax.experimental import pallas as pl
from jax.experimental.pallas import tpu as pltpu
from jax.experimental.pallas import tpu_sc as plsc

_NF = 26
_ED = 16
_FD = 260000
_B = 4096
_NP = (_NF * (_NF - 1)) // 2
_OFF = np.arange(_NF, dtype=np.int32) * 10000

_pairs = np.array([(f, g) for f in range(_NF - 1) for g in range(f + 1, _NF)],
                  dtype=np.int32)
_PF = _pairs[:, 0]
_PG = _pairs[:, 1]

_NW = 32
_RPS = _B // _NW
_RT = 4
_STEPS = _RPS // _RT
_IAB = _RT * 2 * _NP
_CH = 128
_NCH_FULL = _IAB // _CH
_TAIL = _IAB - _NCH_FULL * _CH
_IE = _RT * _NF


def _sc_body(ffm_f, iab_h, ie_h, iq_h, emb_h, lw16_h,
             inter_o, hrows_o, lwg_o,
             iab_v, ie_v, iq_v, buf_v, out_v, ebuf_v, lbuf_v, sem, sem2):
    nc = 2
    wid = lax.axis_index("s") * nc + lax.axis_index("c")

    def step(t, carry):
        row0 = wid * _RPS + t * _RT
        iab_off = pl.multiple_of(row0 * 2 * _NP, _IAB)
        ie_off = pl.multiple_of(row0 * _NF, _IE)
        pltpu.sync_copy(iab_h.at[pl.ds(iab_off, _IAB)], iab_v)
        pltpu.sync_copy(ie_h.at[pl.ds(ie_off, _IE)], ie_v)
        pltpu.sync_copy(iq_h.at[pl.ds(ie_off, _IE)], iq_v)
        cps = []
        for c in range(_NCH_FULL):
            cps.append(pltpu.async_copy(
                ffm_f.at[iab_v.at[pl.ds(c * _CH, _CH)]],
                buf_v.at[pl.ds(c * _CH, _CH)], sem))
        cps.append(pltpu.async_copy(
            ffm_f.at[iab_v.at[pl.ds(_NCH_FULL * _CH, _TAIL)]],
            buf_v.at[pl.ds(_NCH_FULL * _CH, _TAIL)], sem))
        cps.append(pltpu.async_copy(emb_h.at[ie_v], ebuf_v, sem2))
        cps.append(pltpu.async_copy(lw16_h.at[iq_v], lbuf_v, sem2))
        for cp in cps:
            cp.wait()

        def prod(i, c2):
            a = buf_v[2 * i, :]
            b = buf_v[2 * i + 1, :]
            out_v[pl.ds(pl.multiple_of(i * _ED, _ED), _ED)] = a * b
            return c2

        lax.fori_loop(0, _RT * _NP, prod, 0)
        int_off = pl.multiple_of(row0 * _NP * _ED, _RT * _NP * _ED)
        pltpu.sync_copy(out_v, inter_o.at[pl.ds(int_off, _RT * _NP * _ED)])
        pltpu.sync_copy(ebuf_v, hrows_o.at[pl.ds(ie_off, _IE)])
        pltpu.sync_copy(lbuf_v, lwg_o.at[pl.ds(ie_off, _IE)])
        return carry

    lax.fori_loop(0, _STEPS, step, 0)


_sc_call_cache = []


def _sc_call(*args):
    if not _sc_call_cache:
        _sc_call_cache.append(pl.kernel(
            _sc_body,
            out_type=[
                jax.ShapeDtypeStruct((_B * _NP * _ED,), jnp.float32),
                jax.ShapeDtypeStruct((_B * _NF, _ED), jnp.float32),
                jax.ShapeDtypeStruct((_B * _NF, _ED), jnp.float32),
            ],
            mesh=plsc.VectorSubcoreMesh(core_axis_name="c",
                                        subcore_axis_name="s"),
            compiler_params=pltpu.CompilerParams(use_tc_tiling_on_sc=False),
            scratch_types=[
                pltpu.VMEM((_IAB,), jnp.int32),
                pltpu.VMEM((_IE,), jnp.int32),
                pltpu.VMEM((_IE,), jnp.int32),
                pltpu.VMEM((_IAB, _ED), jnp.float32),
                pltpu.VMEM((_RT * _NP * _ED,), jnp.float32),
                pltpu.VMEM((_IE, _ED), jnp.float32),
                pltpu.VMEM((_IE, _ED), jnp.float32),
                pltpu.SemaphoreType.DMA,
                pltpu.SemaphoreType.DMA,
            ],
        ))
    return _sc_call_cache[0](*args)

_BB = 512


def _tc_body(inter_r, h_r, lwr_r, oh_r, lb_r,
             w1_r, b1_r, w2_r, b2_r,
             wf1f_r, wf1i_r, wf1d_r, bf1_r,
             wf2_r, bf2_r, wo_r, bo_r, out_r):
    dot = functools.partial(jnp.dot, preferred_element_type=jnp.float32,
                            precision=lax.Precision.HIGHEST)
    first = (jnp.sum(lwr_r[...] * oh_r[...], axis=1, keepdims=True)
             + lb_r[0, 0])
    d1 = jnp.maximum(dot(h_r[...], w1_r[...]) + b1_r[...], 0.0)
    d2 = jnp.maximum(dot(d1, w2_r[...]) + b2_r[...], 0.0)
    t = (dot(inter_r[...], wf1i_r[...]) + dot(d2, wf1d_r[...])
         + first * wf1f_r[...] + bf1_r[...])
    h2 = jnp.maximum(t, 0.0)
    h3 = jnp.maximum(dot(h2, wf2_r[...]) + bf2_r[...], 0.0)
    out_r[...] = dot(h3, wo_r[...]) + bo_r[...]


def _full(shape):
    return pl.BlockSpec(shape, lambda i: (0, 0))


_tc_call = pl.pallas_call(
    _tc_body,
    grid=(_B // _BB,),
    in_specs=[
        pl.BlockSpec((_BB, _NP * _ED), lambda i: (i, 0)),
        pl.BlockSpec((_BB, _NF * _ED), lambda i: (i, 0)),
        pl.BlockSpec((_BB, _NF * _ED), lambda i: (i, 0)),
        pl.BlockSpec((_BB, _NF * _ED), lambda i: (i, 0)),
        _full((1, 1)),
        _full((_NF * _ED, 64)), _full((1, 64)),
        _full((64, 64)), _full((1, 64)),
        _full((1, 64)), _full((_NP * _ED, 64)), _full((64, 64)), _full((1, 64)),
        _full((64, 32)), _full((1, 32)),
        _full((32, 1)), _full((1, 1)),
    ],
    out_specs=pl.BlockSpec((_BB, 1), lambda i: (i, 0)),
    out_shape=jax.ShapeDtypeStruct((_B, 1), jnp.float32),
)


def _fold_bn(W, b, g, be, m, v):
    s = g * lax.rsqrt(v + 1e-5)
    return W * s[None, :], ((b - m) * s + be)[None, :]


def kernel(x, linear_w, linear_b, emb, ffm,
           W1, b1, g1, be1, m1, v1, W2, b2, g2, be2, m2, v2,
           Wf1, bf1, gf1, bef1, mf1, vf1, Wf2, bf2, gf2, bef2, mf2, vf2,
           Wout, bout):
    xo = x + jnp.asarray(_OFF, dtype=x.dtype)[None, :]
    idx_a = xo[:, _PG] + jnp.asarray(_PF * _FD)[None, :]
    idx_b = xo[:, _PF] + jnp.asarray(_PG * _FD)[None, :]
    iab = jnp.stack([idx_a, idx_b], axis=-1).reshape(-1)
    ie = xo.reshape(-1)
    iq = ie // _ED
    oh = (jnp.arange(_ED, dtype=x.dtype)[None, None, :]
          == (xo % _ED)[:, :, None]).astype(jnp.float32).reshape(_B, _NF * _ED)
    ffm_f = ffm.reshape(_NF * _FD, _ED)
    lw16 = linear_w.reshape(_FD // _ED, _ED)

    inter_f, hrows, lwrows = _sc_call(ffm_f, iab, ie, iq, emb, lw16)

    W1p, b1p = _fold_bn(W1, b1, g1, be1, m1, v1)
    W2p, b2p = _fold_bn(W2, b2, g2, be2, m2, v2)
    Wf1p, bf1p = _fold_bn(Wf1, bf1, gf1, bef1, mf1, vf1)
    Wf2p, bf2p = _fold_bn(Wf2, bf2, gf2, bef2, mf2, vf2)

    out2d = _tc_call(
        inter_f.reshape(_B, _NP * _ED),
        hrows.reshape(_B, _NF * _ED),
        lwrows.reshape(_B, _NF * _ED),
        oh,
        linear_b.reshape(1, 1),
        W1p, b1p, W2p, b2p,
        Wf1p[0:1, :], Wf1p[1:1 + _NP * _ED, :], Wf1p[1 + _NP * _ED:, :], bf1p,
        Wf2p, bf2p, Wout, bout.reshape(1, 1),
    )
    return out2d[:, 0]


# speedup vs baseline: 8.9230x; 1.5109x over previous
import functools

import numpy as np
import jax
import jax.numpy as jnp
from jax import lax
from jax.experimental import pallas as pl
from jax.experimental.pallas import tpu as pltpu
from jax.experimental.pallas import tpu_sc as plsc

_NF = 26
_NFP = 32
_ED = 16
_FD = 260000
_B = 4096
_NP = (_NF * (_NF - 1)) // 2
_NPP = 336
_OFF = np.arange(_NF, dtype=np.int32) * 10000

_pairs = np.array([(f, g) for f in range(_NF - 1) for g in range(f + 1, _NF)],
                  dtype=np.int32)
_PF = _pairs[:, 0]
_PG = _pairs[:, 1]

_NW = 32
_RPS = _B // _NW
_RT = 4
_STEPS = _RPS // _RT
_IAB = _RT * 2 * _NPP
_CH = 128
_NCH = _IAB // _CH
_IE = _RT * _NFP


def _static_tables():
    col1 = np.zeros(2 * _NPP, np.int32)
    add1 = np.zeros(2 * _NPP, np.int32)
    col1[0:650:2] = _PG
    add1[0:650:2] = _OFF[_PG] + _PF * _FD
    col1[1:650:2] = _PF
    add1[1:650:2] = _OFF[_PF] + _PG * _FD
    cols = np.concatenate([col1 + 32 * r for r in range(_RT)])
    adds = np.concatenate([add1 for _ in range(_RT)])
    offc = np.zeros(_NFP, np.int32)
    offc[:_NF] = _OFF
    offs = np.tile(offc, _RT)
    return jnp.asarray(cols), jnp.asarray(adds), jnp.asarray(offs)


def _sc_body(ffm_f, xp_h, cols_h, adds_h, offs_h, emb_h, lw16_h,
             inter_o, hrows_o, lwg_o,
             cols_v, adds_v, offs_v,
             xf_v, iab_v, ie_v, iq_v, buf_v, out_v, ebuf_v, lbuf_v,
             sem, sem2):
    nc = 2
    wid = lax.axis_index("s") * nc + lax.axis_index("c")
    pltpu.sync_copy(cols_h, cols_v)
    pltpu.sync_copy(adds_h, adds_v)
    pltpu.sync_copy(offs_h, offs_v)

    def step(t, carry):
        row0 = wid * _RPS + t * _RT
        x_off = pl.multiple_of(row0 * _NFP, _IE)
        pltpu.sync_copy(xp_h.at[pl.ds(x_off, _IE)], xf_v)

        def bld_ie(i, c2):
            s = pl.ds(pl.multiple_of(i * 16, 16), 16)
            xo = xf_v[s] + offs_v[s]
            ie_v[s] = xo
            iq_v[s] = xo >> 4
            return c2

        lax.fori_loop(0, _IE // 16, bld_ie, 0)

        def bld_iab(i, c2):
            s = pl.ds(pl.multiple_of(i * 16, 16), 16)
            iab_v[s] = plsc.load_gather(xf_v, [cols_v[s]]) + adds_v[s]
            return c2

        lax.fori_loop(0, _IAB // 16, bld_iab, 0)

        cps = []
        for c in range(_NCH):
            cps.append(pltpu.async_copy(
                ffm_f.at[iab_v.at[pl.ds(c * _CH, _CH)]],
                buf_v.at[pl.ds(c * _CH, _CH)], sem))
        cps.append(pltpu.async_copy(emb_h.at[ie_v], ebuf_v, sem2))
        cps.append(pltpu.async_copy(lw16_h.at[iq_v], lbuf_v, sem2))
        for cp in cps:
            cp.wait()

        for r in range(_RT):
            base_in = r * 2 * _NPP
            base_out = r * _NP * _ED

            def prod(i, c2, base_in=base_in, base_out=base_out):
                a = buf_v[base_in + 2 * i, :]
                b = buf_v[base_in + 2 * i + 1, :]
                o = pl.ds(pl.multiple_of(base_out + i * _ED, _ED), _ED)
                out_v[o] = a * b
                return c2

            lax.fori_loop(0, _NP, prod, 0)

        int_off = pl.multiple_of(row0 * _NP * _ED, _RT * _NP * _ED)
        pltpu.sync_copy(out_v, inter_o.at[pl.ds(int_off, _RT * _NP * _ED)])
        ie_off = pl.multiple_of(row0 * _NFP, _IE)
        pltpu.sync_copy(ebuf_v, hrows_o.at[pl.ds(ie_off, _IE)])
        pltpu.sync_copy(lbuf_v, lwg_o.at[pl.ds(ie_off, _IE)])
        return carry

    lax.fori_loop(0, _STEPS, step, 0)


_sc_call_cache = []


def _sc_call(*args):
    if not _sc_call_cache:
        _sc_call_cache.append(pl.kernel(
            _sc_body,
            out_type=[
                jax.ShapeDtypeStruct((_B * _NP * _ED,), jnp.float32),
                jax.ShapeDtypeStruct((_B * _NFP, _ED), jnp.float32),
                jax.ShapeDtypeStruct((_B * _NFP, _ED), jnp.float32),
            ],
            mesh=plsc.VectorSubcoreMesh(core_axis_name="c",
                                        subcore_axis_name="s"),
            compiler_params=pltpu.CompilerParams(use_tc_tiling_on_sc=False,
                                                 needs_layout_passes=False),
            scratch_types=[
                pltpu.VMEM((_IAB,), jnp.int32),
                pltpu.VMEM((_IAB,), jnp.int32),
                pltpu.VMEM((_IE,), jnp.int32),
                pltpu.VMEM((_IE,), jnp.int32),
                pltpu.VMEM((_IAB,), jnp.int32),
                pltpu.VMEM((_IE,), jnp.int32),
                pltpu.VMEM((_IE,), jnp.int32),
                pltpu.VMEM((_IAB, _ED), jnp.float32),
                pltpu.VMEM((_RT * _NP * _ED,), jnp.float32),
                pltpu.VMEM((_IE, _ED), jnp.float32),
                pltpu.VMEM((_IE, _ED), jnp.float32),
                pltpu.SemaphoreType.DMA,
                pltpu.SemaphoreType.DMA,
            ],
        ))
    return _sc_call_cache[0](*args)


_BB = 512


def _tc_body(inter_r, h_r, lwr_r, oh_r, lb_r,
             w1_r, b1_r, w2_r, b2_r,
             wf1f_r, wf1i_r, wf1d_r, bf1_r,
             wf2_r, bf2_r, wo_r, bo_r, out_r):
    dot = functools.partial(jnp.dot, preferred_element_type=jnp.float32,
                            precision=lax.Precision.HIGHEST)
    first = (jnp.sum(lwr_r[...] * oh_r[...], axis=1, keepdims=True)
             + lb_r[0, 0])
    d1 = jnp.maximum(dot(h_r[...], w1_r[...]) + b1_r[...], 0.0)
    d2 = jnp.maximum(dot(d1, w2_r[...]) + b2_r[...], 0.0)
    t = (dot(inter_r[...], wf1i_r[...]) + dot(d2, wf1d_r[...])
         + first * wf1f_r[...] + bf1_r[...])
    h2 = jnp.maximum(t, 0.0)
    h3 = jnp.maximum(dot(h2, wf2_r[...]) + bf2_r[...], 0.0)
    out_r[...] = dot(h3, wo_r[...]) + bo_r[...]


def _full(shape):
    return pl.BlockSpec(shape, lambda i: (0, 0))


_tc_call = pl.pallas_call(
    _tc_body,
    grid=(_B // _BB,),
    in_specs=[
        pl.BlockSpec((_BB, _NP * _ED), lambda i: (i, 0)),
        pl.BlockSpec((_BB, _NFP * _ED), lambda i: (i, 0)),
        pl.BlockSpec((_BB, _NFP * _ED), lambda i: (i, 0)),
        pl.BlockSpec((_BB, _NFP * _ED), lambda i: (i, 0)),
        _full((1, 1)),
        _full((_NFP * _ED, 64)), _full((1, 64)),
        _full((64, 64)), _full((1, 64)),
        _full((1, 64)), _full((_NP * _ED, 64)), _full((64, 64)), _full((1, 64)),
        _full((64, 32)), _full((1, 32)),
        _full((32, 1)), _full((1, 1)),
    ],
    out_specs=pl.BlockSpec((_BB, 1), lambda i: (i, 0)),
    out_shape=jax.ShapeDtypeStruct((_B, 1), jnp.float32),
)


def _fold_bn(W, b, g, be, m, v):
    s = g * lax.rsqrt(v + 1e-5)
    return W * s[None, :], ((b - m) * s + be)[None, :]


def kernel(x, linear_w, linear_b, emb, ffm,
           W1, b1, g1, be1, m1, v1, W2, b2, g2, be2, m2, v2,
           Wf1, bf1, gf1, bef1, mf1, vf1, Wf2, bf2, gf2, bef2, mf2, vf2,
           Wout, bout):
    xp = jnp.pad(x, ((0, 0), (0, _NFP - _NF))).reshape(-1)
    xo = x + jnp.asarray(_OFF, dtype=x.dtype)[None, :]
    ohsmall = (jnp.arange(_ED, dtype=x.dtype)[None, None, :]
               == (xo % _ED)[:, :, None]).astype(jnp.float32)
    oh = jnp.pad(ohsmall, ((0, 0), (0, _NFP - _NF), (0, 0)))
    oh = oh.reshape(_B, _NFP * _ED)
    cols, adds, offs = _static_tables()
    ffm_f = ffm.reshape(_NF * _FD, _ED)
    lw16 = linear_w.reshape(_FD // _ED, _ED)

    inter_f, hrows, lwrows = _sc_call(ffm_f, xp, cols, adds, offs,
                                      emb, lw16)

    W1p, b1p = _fold_bn(W1, b1, g1, be1, m1, v1)
    W1a = jnp.pad(W1p.reshape(_NF, _ED, 64),
                  ((0, _NFP - _NF), (0, 0), (0, 0))).reshape(_NFP * _ED, 64)
    W2p, b2p = _fold_bn(W2, b2, g2, be2, m2, v2)
    Wf1p, bf1p = _fold_bn(Wf1, bf1, gf1, bef1, mf1, vf1)
    Wf2p, bf2p = _fold_bn(Wf2, bf2, gf2, bef2, mf2, vf2)

    out2d = _tc_call(
        inter_f.reshape(_B, _NP * _ED),
        hrows.reshape(_B, _NFP * _ED),
        lwrows.reshape(_B, _NFP * _ED),
        oh,
        linear_b.reshape(1, 1),
        W1a, b1p, W2p, b2p,
        Wf1p[0:1, :], Wf1p[1:1 + _NP * _ED, :], Wf1p[1 + _NP * _ED:, :], bf1p,
        Wf2p, bf2p, Wout, bout.reshape(1, 1),
    )
    return out2d[:, 0]
